# baseline (speedup 1.0000x reference)
import jax
import jax.numpy as jnp
from jax import lax
from jax.experimental import pallas as pl
from jax.experimental.pallas import tpu as pltpu
from jax.experimental.pallas import tpu_sc as plsc

NUM_ENT = 1000000
NUM_REL = 1000
EMB_DIM = 64
BATCH = 16384
GAMMA = 2.0

NC = 2
NS = 16
L = 16
NW = NC * NS
B_PER_W = BATCH // NW
CHUNK = 16
NCHUNK = B_PER_W // CHUNK
ROWG = 8

_PERM_DNUMS = lax.GatherDimensionNumbers(
    offset_dims=(), collapsed_slice_dims=(0,), start_index_map=(0,))


def _lane_perm(x, idx):
    return lax.gather(x, idx[:, None], _PERM_DNUMS, slice_sizes=(1,),
                      mode=lax.GatherScatterMode.PROMISE_IN_BOUNDS)


def _vsqrt(x):
    y = jnp.where(x < 0.0125, jnp.float32(0.05),
        jnp.where(x < 0.125, jnp.float32(0.2),
        jnp.where(x < 0.7, jnp.float32(0.54), jnp.float32(1.12))))
    for _ in range(5):
        y = 0.5 * (y + x / y)
    return jnp.where(x < 1e-12, jnp.float32(0.0), y)


def _body(hs_hbm, rs_hbm, ts_hbm, ent_hbm, rel_hbm, out_hbm,
          idx_h, idx_r, idx_t,
          h0, r0, t0, h1, r1, t1, out_v, sem0, sem1):
    wid = lax.axis_index("s") * NC + lax.axis_index("c")
    base = wid * B_PER_W

    pltpu.sync_copy(hs_hbm.at[pl.ds(base, B_PER_W)], idx_h)
    pltpu.sync_copy(rs_hbm.at[pl.ds(base, B_PER_W)], idx_r)
    pltpu.sync_copy(ts_hbm.at[pl.ds(base, B_PER_W)], idx_t)

    lane = lax.iota(jnp.int32, L)
    bufs = ((h0, r0, t0, sem0), (h1, r1, t1, sem1))

    def fire(ch, bset):
        hb, rb, tb, sem = bset
        sl = pl.ds(ch * CHUNK, L)
        ivh = idx_h[sl]
        ivr = idx_r[sl]
        ivt = idx_t[sl]
        for j in range(L):
            bh = pl.multiple_of((ivh[j] >> 3) * ROWG, ROWG)
            br = pl.multiple_of((ivr[j] >> 3) * ROWG, ROWG)
            bt = pl.multiple_of((ivt[j] >> 3) * ROWG, ROWG)
            pltpu.async_copy(ent_hbm.at[pl.ds(bh, ROWG), :], hb.at[j], sem)
            pltpu.async_copy(rel_hbm.at[pl.ds(br, ROWG), :], rb.at[j], sem)
            pltpu.async_copy(ent_hbm.at[pl.ds(bt, ROWG), :], tb.at[j], sem)

    def drain_compute(ch, bset):
        hb, rb, tb, sem = bset
        for j in range(L):
            pltpu.make_async_copy(
                ent_hbm.at[pl.ds(0, ROWG), :], hb.at[j], sem).wait()
            pltpu.make_async_copy(
                rel_hbm.at[pl.ds(0, ROWG), :], rb.at[j], sem).wait()
            pltpu.make_async_copy(
                ent_hbm.at[pl.ds(0, ROWG), :], tb.at[j], sem).wait()
        sl = pl.ds(ch * CHUNK, L)
        mh = idx_h[sl] & 7
        mr = idx_r[sl] & 7
        mt = idx_t[sl] & 7
        sums = jnp.zeros((L,), jnp.float32)
        for j in range(L):
            acc = jnp.zeros((L,), jnp.float32)
            for c in range(EMB_DIM // L):
                csl = pl.ds(c * L, L)
                d = hb[j, mh[j], csl] + rb[j, mr[j], csl] - tb[j, mt[j], csl]
                acc = acc + d * d
            for k in (8, 4, 2, 1):
                acc = acc + _lane_perm(acc, lane ^ k)
            sums = jnp.where(lane == j, acc, sums)
        out_v[sl] = GAMMA - _vsqrt(sums)

    fire(0, bufs[0])

    def step(i, carry):
        fire(2 * i + 1, bufs[1])
        drain_compute(2 * i, bufs[0])

        @pl.when(i < NCHUNK // 2 - 1)
        def _():
            fire(2 * i + 2, bufs[0])

        drain_compute(2 * i + 1, bufs[1])
        return carry

    lax.fori_loop(0, NCHUNK // 2, step, 0)

    pltpu.sync_copy(out_v, out_hbm.at[pl.ds(base, B_PER_W)])


@jax.jit
def _transe(hs, rs, ts, ent_embs, rel_embs):
    mesh = plsc.VectorSubcoreMesh(
        core_axis_name="c", subcore_axis_name="s",
        num_cores=NC, num_subcores=NS)
    buf = pltpu.VMEM((CHUNK, ROWG, EMB_DIM), jnp.float32)
    run = pl.kernel(
        _body,
        out_type=jax.ShapeDtypeStruct((BATCH,), jnp.float32),
        mesh=mesh,
        scratch_types=[
            pltpu.VMEM((B_PER_W,), jnp.int32),
            pltpu.VMEM((B_PER_W,), jnp.int32),
            pltpu.VMEM((B_PER_W,), jnp.int32),
            buf, buf, buf, buf, buf, buf,
            pltpu.VMEM((B_PER_W,), jnp.float32),
            pltpu.SemaphoreType.DMA,
            pltpu.SemaphoreType.DMA,
        ],
    )
    return run(hs, rs, ts, ent_embs, rel_embs)


def kernel(hs, rs, ts, ent_embs, rel_embs):
    score = _transe(hs.astype(jnp.int32), rs.astype(jnp.int32),
                    ts.astype(jnp.int32), ent_embs, rel_embs)
    return score.reshape(-1, 1)
